# Initial kernel scaffold; baseline (speedup 1.0000x reference)
import jax
import jax.numpy as jnp
from jax.experimental import pallas as pl


def kernel(board_input, hash_features, W1, b1, W2, b2, W3, b3):
    raise NotImplementedError("write your pallas kernel here")


# speedup vs baseline: 29.9779x; 29.9779x over previous
import functools

import jax
import jax.numpy as jnp
import numpy as np
from jax import lax
from jax.experimental import pallas as pl
from jax.experimental.pallas import tpu as pltpu
from jax.experimental.pallas import tpu_sc as plsc

K = 3
DIM_FEATURE = 32
B = 4096
Hb = 15
Wb = 15
C = 2
H = Hb - K + 1
P = H * H
NROWS = B * P
NW = 32
BOARDS_PER_W = B // NW
CHUNK = 128
CHUNKS_PER_W = BOARDS_PER_W * P // CHUNK
QMAX = 127.0 / 128.0


def _build_unfold_matrix():
    m = np.zeros((C * Hb * Wb, P), dtype=np.float32)
    for c in range(C):
        for di in range(K):
            for dj in range(K):
                w = float(2 ** (c * 9 + di * 3 + dj))
                for i in range(H):
                    for j in range(H):
                        m[c * Hb * Wb + (i + di) * Wb + (j + dj), i * H + j] += w
    return m


_M_CONST = _build_unfold_matrix()
_BIDX_CONST = ((np.arange(NROWS, dtype=np.int64) // P) % (B // 2)).astype(
    np.int32).reshape(NROWS // CHUNK, CHUNK)
_ZERO_ROWS = np.zeros((CHUNK, DIM_FEATURE), dtype=np.float32)


def _index_body(x_ref, m_ref, o_ref):
    acc = jax.lax.dot_general(
        x_ref[...], m_ref[...], (((1,), (0,)), ((), ())),
        preferred_element_type=jnp.float32)
    o_ref[...] = acc.astype(jnp.int32)


def _compute_indices(board_flat, m):
    blk = 512
    return pl.pallas_call(
        _index_body,
        grid=(B // blk,),
        in_specs=[
            pl.BlockSpec((blk, C * Hb * Wb), lambda i: (i, 0)),
            pl.BlockSpec((C * Hb * Wb, P), lambda i: (0, 0)),
        ],
        out_specs=pl.BlockSpec((blk, P), lambda i: (i, 0)),
        out_shape=jax.ShapeDtypeStruct((B, P), jnp.int32),
    )(board_flat, m)


def _quant_body(t_ref, o_ref):
    x = jnp.clip(t_ref[...], -1.0, QMAX)
    o_ref[...] = jnp.round(x * 128.0) * (1.0 / 128.0)


def _quantize_table(table):
    v, d = table.shape
    wide = table.reshape(v * d // 128, 128)
    blk = 8192
    q = pl.pallas_call(
        _quant_body,
        grid=(wide.shape[0] // blk,),
        in_specs=[pl.BlockSpec((blk, 128), lambda i: (i, 0))],
        out_specs=pl.BlockSpec((blk, 128), lambda i: (i, 0)),
        out_shape=jax.ShapeDtypeStruct(wide.shape, jnp.float32),
    )(wide)
    return q.reshape(v, d)


def _sc_gather_sum(qtable, idx2d, bidx2d, zrows):
    mesh = plsc.VectorSubcoreMesh(core_axis_name="c", subcore_axis_name="s")

    @functools.partial(
        pl.kernel,
        out_type=jax.ShapeDtypeStruct((B, DIM_FEATURE), jnp.float32),
        mesh=mesh,
        compiler_params=pltpu.CompilerParams(use_tc_tiling_on_sc=False),
        scratch_types=[
            pltpu.VMEM((CHUNKS_PER_W + 7, CHUNK), jnp.int32),
            pltpu.VMEM((CHUNKS_PER_W + 7, CHUNK), jnp.int32),
            pltpu.VMEM((2, CHUNK, DIM_FEATURE), jnp.float32),
            pltpu.VMEM_SHARED((B // 2, DIM_FEATURE), jnp.float32),
            pltpu.SemaphoreType.DMA,
        ],
    )
    def sck(qtab_hbm, idx_hbm, bidx_hbm, zero_hbm, out_hbm,
            idx_v, bidx_v, rows_v, acc, gsem):
        c = lax.axis_index("c")
        s = lax.axis_index("s")
        wid = c * 16 + s
        cbase = wid * CHUNKS_PER_W
        cbase_al = pl.multiple_of((cbase // 8) * 8, 8)
        off = cbase - cbase_al
        pltpu.sync_copy(idx_hbm.at[pl.ds(cbase_al, CHUNKS_PER_W + 7)], idx_v)
        pltpu.sync_copy(bidx_hbm.at[pl.ds(cbase_al, CHUNKS_PER_W + 7)], bidx_v)
        pltpu.sync_copy(zero_hbm, acc.at[pl.ds(s * BOARDS_PER_W, BOARDS_PER_W)])

        def body(j, carry):
            pltpu.async_copy(qtab_hbm.at[idx_v.at[off + j]], rows_v.at[0], gsem).wait()
            pltpu.sync_copy(rows_v.at[0], acc.at[bidx_v.at[off + j]], add=True)
            return carry

        lax.fori_loop(0, CHUNKS_PER_W, body, 0)
        pltpu.sync_copy(acc.at[pl.ds(s * BOARDS_PER_W, BOARDS_PER_W)],
                        out_hbm.at[pl.ds(wid * BOARDS_PER_W, BOARDS_PER_W)])

    return sck(qtable, idx2d, bidx2d, zrows)


def _mlp_body(f_ref, w1_ref, b1_ref, w2_ref, b2_ref, w3_ref, b3_ref, o_ref):
    def wq(w):
        return jnp.clip(jnp.round(w * 128.0), -128.0, 127.0) * (1.0 / 128.0)

    def bq(b):
        return jnp.round(b * 16384.0) * (1.0 / 16384.0)

    v = jnp.clip(f_ref[...], -1.0, QMAX)
    v = jax.lax.dot_general(v, wq(w1_ref[...]), (((1,), (1,)), ((), ())),
                            preferred_element_type=jnp.float32) + bq(b1_ref[...])
    v = jnp.clip(v, 0.0, QMAX)
    v = jax.lax.dot_general(v, wq(w2_ref[...]), (((1,), (1,)), ((), ())),
                            preferred_element_type=jnp.float32) + bq(b2_ref[...])
    v = jnp.clip(v, 0.0, QMAX)
    v = jax.lax.dot_general(v, wq(w3_ref[...]), (((1,), (1,)), ((), ())),
                            preferred_element_type=jnp.float32) + bq(b3_ref[...])
    o_ref[...] = v


def _mlp(feature, W1, b1, W2, b2, W3, b3):
    return pl.pallas_call(
        _mlp_body,
        out_shape=jax.ShapeDtypeStruct((B, 3), jnp.float32),
    )(feature, W1, b1.reshape(1, -1), W2, b2.reshape(1, -1),
      W3, b3.reshape(1, -1))


def kernel(board_input, hash_features, W1, b1, W2, b2, W3, b3):
    board_flat = board_input.reshape(B, C * Hb * Wb)
    idx = _compute_indices(board_flat, jnp.asarray(_M_CONST))
    qtab = _quantize_table(hash_features)
    feature = _sc_gather_sum(qtab, idx.reshape(NROWS // CHUNK, CHUNK),
                             jnp.asarray(_BIDX_CONST), jnp.asarray(_ZERO_ROWS))
    value = _mlp(feature, W1, b1, W2, b2, W3, b3)
    policy = jnp.zeros((B, Hb, Wb), dtype=jnp.float32)
    return (value, policy)


# speedup vs baseline: 37.7203x; 1.2583x over previous
import functools

import jax
import jax.numpy as jnp
import numpy as np
from jax import lax
from jax.experimental import pallas as pl
from jax.experimental.pallas import tpu as pltpu
from jax.experimental.pallas import tpu_sc as plsc

K = 3
DIM_FEATURE = 32
B = 4096
Hb = 15
Wb = 15
C = 2
H = Hb - K + 1
P = H * H
NROWS = B * P
NW = 32
BOARDS_PER_W = B // NW
CHUNK = 128
CHUNKS_PER_W = BOARDS_PER_W * P // CHUNK
QMAX = 127.0 / 128.0


def _build_unfold_matrix():
    m = np.zeros((C * Hb * Wb, P), dtype=np.float32)
    for c in range(C):
        for di in range(K):
            for dj in range(K):
                w = float(2 ** (c * 9 + di * 3 + dj))
                for i in range(H):
                    for j in range(H):
                        m[c * Hb * Wb + (i + di) * Wb + (j + dj), i * H + j] += w
    return m


_M_CONST = _build_unfold_matrix()
_BIDX_CONST = ((np.arange(NROWS, dtype=np.int64) // P) % (B // 2)).astype(
    np.int32).reshape(NROWS // CHUNK, CHUNK)
_ZERO_ROWS = np.zeros((CHUNK, DIM_FEATURE), dtype=np.float32)


def _index_body(x_ref, m_ref, o_ref):
    acc = jax.lax.dot_general(
        x_ref[...], m_ref[...], (((1,), (0,)), ((), ())),
        preferred_element_type=jnp.float32)
    o_ref[...] = acc.astype(jnp.int32)


def _compute_indices(board_flat, m):
    blk = 512
    return pl.pallas_call(
        _index_body,
        grid=(B // blk,),
        in_specs=[
            pl.BlockSpec((blk, C * Hb * Wb), lambda i: (i, 0)),
            pl.BlockSpec((C * Hb * Wb, P), lambda i: (0, 0)),
        ],
        out_specs=pl.BlockSpec((blk, P), lambda i: (i, 0)),
        out_shape=jax.ShapeDtypeStruct((B, P), jnp.int32),
    )(board_flat, m)


def _quant_body(t_ref, o_ref):
    x = jnp.clip(t_ref[...], -1.0, QMAX)
    o_ref[...] = jnp.round(x * 128.0) * (1.0 / 128.0)


def _quantize_table(table):
    v, d = table.shape
    wide = table.reshape(v * d // 128, 128)
    blk = 8192
    q = pl.pallas_call(
        _quant_body,
        grid=(wide.shape[0] // blk,),
        in_specs=[pl.BlockSpec((blk, 128), lambda i: (i, 0))],
        out_specs=pl.BlockSpec((blk, 128), lambda i: (i, 0)),
        out_shape=jax.ShapeDtypeStruct(wide.shape, jnp.float32),
    )(wide)
    return q.reshape(v, d)


def _sc_gather_sum(qtable, idx2d, bidx2d, zrows):
    mesh = plsc.VectorSubcoreMesh(core_axis_name="c", subcore_axis_name="s")

    @functools.partial(
        pl.kernel,
        out_type=jax.ShapeDtypeStruct((B, DIM_FEATURE), jnp.float32),
        mesh=mesh,
        compiler_params=pltpu.CompilerParams(use_tc_tiling_on_sc=False),
        scratch_types=[
            pltpu.VMEM((CHUNKS_PER_W + 7, CHUNK), jnp.int32),
            pltpu.VMEM((CHUNKS_PER_W + 7, CHUNK), jnp.int32),
            pltpu.VMEM((4, CHUNK, DIM_FEATURE), jnp.float32),
            pltpu.VMEM_SHARED((B // 2, DIM_FEATURE), jnp.float32),
            pltpu.SemaphoreType.DMA,
            pltpu.SemaphoreType.DMA,
            pltpu.SemaphoreType.DMA,
            pltpu.SemaphoreType.DMA,
        ],
    )
    def sck(qtab_hbm, idx_hbm, bidx_hbm, zero_hbm, out_hbm,
            idx_v, bidx_v, rows_v, acc, g0, g1, g2, g3):
        c = lax.axis_index("c")
        s = lax.axis_index("s")
        wid = c * 16 + s
        cbase = wid * CHUNKS_PER_W
        cbase_al = pl.multiple_of((cbase // 8) * 8, 8)
        off = cbase - cbase_al
        pltpu.sync_copy(idx_hbm.at[pl.ds(cbase_al, CHUNKS_PER_W + 7)], idx_v)
        pltpu.sync_copy(bidx_hbm.at[pl.ds(cbase_al, CHUNKS_PER_W + 7)], bidx_v)
        pltpu.sync_copy(zero_hbm, acc.at[pl.ds(s * BOARDS_PER_W, BOARDS_PER_W)])

        def gissue(j, b, sem):
            pltpu.async_copy(qtab_hbm.at[idx_v.at[off + j]], rows_v.at[b], sem)

        def gwait(j, b, sem):
            pltpu.make_async_copy(qtab_hbm.at[idx_v.at[off + j]],
                                  rows_v.at[b], sem).wait()

        def scat(j, b):
            pltpu.sync_copy(rows_v.at[b], acc.at[bidx_v.at[off + j]], add=True)

        gissue(0, 0, g0)

        def body(t, carry):
            j = 2 * t
            gissue(j + 1, 1, g1)
            gwait(j, 0, g0)
            scat(j, 0)
            gissue(j + 2, 0, g0)
            gwait(j + 1, 1, g1)
            scat(j + 1, 1)
            return carry

        lax.fori_loop(0, (CHUNKS_PER_W - 1) // 2, body, 0)
        jlast = CHUNKS_PER_W - 1
        gwait(jlast, 0, g0)
        scat(jlast, 0)
        pltpu.sync_copy(acc.at[pl.ds(s * BOARDS_PER_W, BOARDS_PER_W)],
                        out_hbm.at[pl.ds(wid * BOARDS_PER_W, BOARDS_PER_W)])

    return sck(qtable, idx2d, bidx2d, zrows)


def _mlp_body(f_ref, w1_ref, b1_ref, w2_ref, b2_ref, w3_ref, b3_ref, o_ref):
    def wq(w):
        return jnp.clip(jnp.round(w * 128.0), -128.0, 127.0) * (1.0 / 128.0)

    def bq(b):
        return jnp.round(b * 16384.0) * (1.0 / 16384.0)

    v = jnp.clip(f_ref[...], -1.0, QMAX)
    v = jax.lax.dot_general(v, wq(w1_ref[...]), (((1,), (1,)), ((), ())),
                            preferred_element_type=jnp.float32) + bq(b1_ref[...])
    v = jnp.clip(v, 0.0, QMAX)
    v = jax.lax.dot_general(v, wq(w2_ref[...]), (((1,), (1,)), ((), ())),
                            preferred_element_type=jnp.float32) + bq(b2_ref[...])
    v = jnp.clip(v, 0.0, QMAX)
    v = jax.lax.dot_general(v, wq(w3_ref[...]), (((1,), (1,)), ((), ())),
                            preferred_element_type=jnp.float32) + bq(b3_ref[...])
    o_ref[...] = v


def _mlp(feature, W1, b1, W2, b2, W3, b3):
    return pl.pallas_call(
        _mlp_body,
        out_shape=jax.ShapeDtypeStruct((B, 3), jnp.float32),
    )(feature, W1, b1.reshape(1, -1), W2, b2.reshape(1, -1),
      W3, b3.reshape(1, -1))


def kernel(board_input, hash_features, W1, b1, W2, b2, W3, b3):
    board_flat = board_input.reshape(B, C * Hb * Wb)
    idx = _compute_indices(board_flat, jnp.asarray(_M_CONST))
    qtab = _quantize_table(hash_features)
    feature = _sc_gather_sum(qtab, idx.reshape(NROWS // CHUNK, CHUNK),
                             jnp.asarray(_BIDX_CONST), jnp.asarray(_ZERO_ROWS))
    value = _mlp(feature, W1, b1, W2, b2, W3, b3)
    policy = jnp.zeros((B, Hb, Wb), dtype=jnp.float32)
    return (value, policy)
